# baseline (speedup 1.0000x reference)
import jax
import jax.numpy as jnp
from jax.experimental import pallas as pl

T, D_MODEL, D_FF, E, TOP_K = 32, 768, 1536, 8, 2


def _moe_body(x_ref, ids_ref, tw_ref, w1_ref, w3_ref, w2_ref, out_ref):
    e = pl.program_id(0)

    x = x_ref[...]
    h1 = jax.lax.dot_general(
        x, w1_ref[0, 0], (((1,), (1,)), ((), ())),
        preferred_element_type=jnp.float32)
    h3 = jax.lax.dot_general(
        x, w3_ref[0, 0], (((1,), (1,)), ((), ())),
        preferred_element_type=jnp.float32)
    act = h1 * jax.nn.sigmoid(h1) * h3
    o = jax.lax.dot_general(
        act, w2_ref[0], (((1,), (1,)), ((), ())),
        preferred_element_type=jnp.float32)

    gate = jnp.sum(
        jnp.where(ids_ref[...] == e, tw_ref[...], 0.0),
        axis=1, keepdims=True)

    @pl.when(e == 0)
    def _first():
        out_ref[...] = gate * o

    @pl.when(e > 0)
    def _rest():
        out_ref[...] += gate * o


@jax.jit
def kernel(x, topk_ids, topk_weight, w13_weight, w2_weight):
    w13 = w13_weight.reshape(E, 2, D_FF, D_MODEL)
    return pl.pallas_call(
        _moe_body,
        grid=(E,),
        in_specs=[
            pl.BlockSpec((T, D_MODEL), lambda e: (0, 0)),
            pl.BlockSpec((T, TOP_K), lambda e: (0, 0)),
            pl.BlockSpec((T, TOP_K), lambda e: (0, 0)),
            pl.BlockSpec((1, 1, D_FF, D_MODEL), lambda e: (e, 0, 0, 0)),
            pl.BlockSpec((1, 1, D_FF, D_MODEL), lambda e: (e, 1, 0, 0)),
            pl.BlockSpec((1, D_MODEL, D_FF), lambda e: (e, 0, 0)),
        ],
        out_specs=pl.BlockSpec((T, D_MODEL), lambda e: (0, 0)),
        out_shape=jax.ShapeDtypeStruct((T, D_MODEL), jnp.float32),
    )(x, topk_ids, topk_weight, w13, w13, w2_weight)


# speedup vs baseline: 1.0447x; 1.0447x over previous
import jax
import jax.numpy as jnp
from jax.experimental import pallas as pl

T, D_MODEL, D_FF, E, TOP_K = 32, 768, 1536, 8, 2


def _moe_body(x_ref, ids_ref, tw_ref, w1_ref, w3_ref, w2_ref, out_ref):
    e = pl.program_id(0)

    @pl.when(e == 0)
    def _init():
        out_ref[...] = jnp.zeros_like(out_ref)

    x = x_ref[...]
    h1 = jax.lax.dot_general(
        x, w1_ref[0, 0], (((1,), (1,)), ((), ())),
        preferred_element_type=jnp.float32)
    h3 = jax.lax.dot_general(
        x, w3_ref[0, 0], (((1,), (1,)), ((), ())),
        preferred_element_type=jnp.float32)
    act = h1 * jax.nn.sigmoid(h1) * h3
    o = jax.lax.dot_general(
        act, w2_ref[0], (((1,), (1,)), ((), ())),
        preferred_element_type=jnp.float32)

    gate = jnp.sum(
        jnp.where(ids_ref[...] == e, tw_ref[...], 0.0),
        axis=1, keepdims=True)
    out_ref[...] += gate * o


@jax.jit
def kernel(x, topk_ids, topk_weight, w13_weight, w2_weight):
    w13 = w13_weight.reshape(E, 2, D_FF, D_MODEL)
    return pl.pallas_call(
        _moe_body,
        grid=(E,),
        in_specs=[
            pl.BlockSpec((T, D_MODEL), lambda e: (0, 0)),
            pl.BlockSpec((T, TOP_K), lambda e: (0, 0)),
            pl.BlockSpec((T, TOP_K), lambda e: (0, 0)),
            pl.BlockSpec((1, 1, D_FF, D_MODEL), lambda e: (e, 0, 0, 0)),
            pl.BlockSpec((1, 1, D_FF, D_MODEL), lambda e: (e, 1, 0, 0)),
            pl.BlockSpec((1, D_MODEL, D_FF), lambda e: (e, 0, 0)),
        ],
        out_specs=pl.BlockSpec((T, D_MODEL), lambda e: (0, 0)),
        out_shape=jax.ShapeDtypeStruct((T, D_MODEL), jnp.float32),
    )(x, topk_ids, topk_weight, w13, w13, w2_weight)
